# baseline (speedup 1.0000x reference)
import jax
import jax.numpy as jnp
from jax import lax
from jax.experimental import pallas as pl
from jax.experimental.pallas import tpu as pltpu
from jax.experimental.pallas import tpu_sc as plsc

_D = 64
_V = 100000
_NI = 4096
_NJ = 200
_NC, _NS = 2, 16
_NW = _NC * _NS
_C = 128
_NBUF = 5
_K = 4
_NGROUP = _NJ // _NBUF


_BC = 2048
_H = 51200


def _sum_body(a0, b0, a1, b1, a2, b2, a3, b3, o):
    sa = (a0[...] + a1[...]) + (a2[...] + a3[...])
    sb = (b0[...] + b1[...]) + (b2[...] + b3[...])
    o[...] = jnp.concatenate([sa.T, sb.T], axis=1)


def _sum_tables(w0t, w1t, w2t, w3t):
    aspec = pl.BlockSpec((_D, _BC), lambda i: (0, i))
    bspec = pl.BlockSpec(
        (_D, _BC), lambda i: (0, jnp.minimum(i + _H // _BC, _V // _BC))
    )
    ospec = pl.BlockSpec((_BC, 2 * _D), lambda i: (i, 0))
    return pl.pallas_call(
        _sum_body, grid=(_H // _BC,),
        in_specs=[aspec, bspec] * 4, out_specs=ospec,
        out_shape=jax.ShapeDtypeStruct((_H, 2 * _D), jnp.float32),
    )(w0t, w0t, w1t, w1t, w2t, w2t, w3t, w3t)


def _gather_body(w_hbm, xt_hbm, o_hbm, idx_v, rows_v, tile_v, *sems):
    gsem, ssem = sems[:_NBUF], sems[_NBUF:]
    wid = lax.axis_index("s") * _NC + lax.axis_index("c")
    iot = lax.iota(jnp.int32, 16)
    dh_base = lax.shift_right_logical(iot, 3)
    dl_vec = lax.bitwise_and(iot, 7)
    pltpu.sync_copy(xt_hbm.at[:, wid], idx_v)

    def remap(t, carry):
        jh = t // 64
        q = t % 64
        v = idx_v[jh, q // 8, pl.ds((q % 8) * 16, 16)]
        g = jnp.where(v < _H, v + v, v + v - (2 * _H - 1))
        idx_v[jh, q // 8, pl.ds((q % 8) * 16, 16)] = g
        return carry

    lax.fori_loop(0, _NJ * 8, remap, 0)

    def start_gather(j, b):
        pltpu.async_copy(
            w_hbm.at[idx_v.at[j // 8, j % 8]], rows_v.at[b], gsem[b]
        )

    def wait_gather(b):
        pltpu.make_async_copy(
            w_hbm.at[pl.ds(0, _C)], rows_v.at[b], gsem[b]
        ).wait()

    def start_store(j, b):
        pltpu.async_copy(
            tile_v.at[b, :, :, pl.ds(0, _C)], o_hbm.at[j, :, wid], ssem[b]
        )

    def wait_store(b):
        pltpu.make_async_copy(
            tile_v.at[b, :, :, pl.ds(0, _C)], o_hbm.at[0, :, wid], ssem[b]
        ).wait()

    def transpose_chunk(b):
        rb, tb = rows_v.at[b], tile_v.at[b]

        def il_loop(t, ilv):
            for u in range(2):
                il = t * 2 + u
                for q in range(4):
                    v = rb[il, pl.ds(16 * q, 16)]
                    plsc.store_scatter(
                        tb, [dh_base + 2 * q, dl_vec, ilv + u], v
                    )
            return ilv + 2

        lax.fori_loop(0, _C // 2, il_loop, iot * 0)

    def substep(j, b, do_prefetch, do_store_wait):
        if do_prefetch:
            start_gather(j + _K, (b + _K) % _NBUF)
        wait_gather(b)
        if do_store_wait:
            wait_store(b)
        transpose_chunk(b)
        start_store(j, b)

    for b in range(_K):
        start_gather(b, b)

    for b in range(_NBUF):
        substep(b, b, True, False)

    def group(g, carry):
        for b in range(_NBUF):
            substep(g * _NBUF + b, b, True, True)
        return carry

    lax.fori_loop(1, _NGROUP - 1, group, 0)

    for b in range(_NBUF):
        substep((_NGROUP - 1) * _NBUF + b, b, b < _NBUF - _K, True)

    for b in range(_NBUF):
        wait_store(b)


_mesh = plsc.VectorSubcoreMesh(
    core_axis_name="c", subcore_axis_name="s",
    num_cores=_NC, num_subcores=_NS,
)

_gather = pl.kernel(
    _gather_body,
    out_type=jax.ShapeDtypeStruct((_NJ, 8, _NW, 8, _C), jnp.float32),
    mesh=_mesh,
    scratch_types=[
        pltpu.VMEM((_NJ // 8, 8, _C), jnp.int32),
        pltpu.VMEM((_NBUF, _C, _D), jnp.float32),
        pltpu.VMEM((_NBUF, 8, 8, 129), jnp.float32),
        *([pltpu.SemaphoreType.DMA] * (2 * _NBUF)),
    ],
    compiler_params=pltpu.CompilerParams(
        use_tc_tiling_on_sc=False, needs_layout_passes=False,
    ),
)


def kernel(x, W0, W1, W2, W3):
    wsum = _sum_tables(W0.T, W1.T, W2.T, W3.T).reshape(2 * _H, _D)
    xt4 = x.T.reshape(_NJ // 8, 8, _NW, _C).transpose(0, 2, 1, 3)
    out5 = _gather(wsum, xt4)
    return out5.transpose(2, 4, 0, 1, 3).reshape(_NI, _NJ, _D)


# speedup vs baseline: 1.0040x; 1.0040x over previous
import jax
import jax.numpy as jnp
from jax import lax
from jax.experimental import pallas as pl
from jax.experimental.pallas import tpu as pltpu
from jax.experimental.pallas import tpu_sc as plsc

_D = 64
_V = 100000
_NI = 4096
_NJ = 200
_NC, _NS = 2, 16
_NW = _NC * _NS
_C = 128
_NBUF = 5
_K = 4
_NGROUP = _NJ // _NBUF


_BC = 2048
_H = 51200


def _sum_body(a0, b0, a1, b1, a2, b2, a3, b3, o):
    sa = (a0[...] + a1[...]) + (a2[...] + a3[...])
    sb = (b0[...] + b1[...]) + (b2[...] + b3[...])
    o[...] = jnp.concatenate([sa.T, sb.T], axis=1)


def _sum_tables(w0t, w1t, w2t, w3t):
    aspec = pl.BlockSpec((_D, _BC), lambda i: (0, i))
    bspec = pl.BlockSpec(
        (_D, _BC), lambda i: (0, jnp.minimum(i + _H // _BC, _V // _BC))
    )
    ospec = pl.BlockSpec((_BC, 2 * _D), lambda i: (i, 0))
    return pl.pallas_call(
        _sum_body, grid=(_H // _BC,),
        in_specs=[aspec, bspec] * 4, out_specs=ospec,
        out_shape=jax.ShapeDtypeStruct((_H, 2 * _D), jnp.float32),
    )(w0t, w0t, w1t, w1t, w2t, w2t, w3t, w3t)


def _gather_body(w_hbm, xt_hbm, o_hbm, idx_v, rows_v, tile_v, *sems):
    gsem, ssem = sems[:_NBUF], sems[_NBUF:]
    wid = lax.axis_index("s") * _NC + lax.axis_index("c")
    iot = lax.iota(jnp.int32, 16)
    dh_base = lax.shift_right_logical(iot, 3)
    dl_vec = lax.bitwise_and(iot, 7)
    qv = [dh_base + 2 * q for q in range(4)]
    pltpu.sync_copy(xt_hbm.at[:, wid], idx_v)

    def remap(t, carry):
        jh = t // 64
        q = t % 64
        v = idx_v[jh, q // 8, pl.ds((q % 8) * 16, 16)]
        g = jnp.where(v < _H, v + v, v + v - (2 * _H - 1))
        idx_v[jh, q // 8, pl.ds((q % 8) * 16, 16)] = g
        return carry

    lax.fori_loop(0, _NJ * 8, remap, 0)

    def start_gather(j, b):
        pltpu.async_copy(
            w_hbm.at[idx_v.at[j // 8, j % 8]], rows_v.at[b], gsem[b]
        )

    def wait_gather(b):
        pltpu.make_async_copy(
            w_hbm.at[pl.ds(0, _C)], rows_v.at[b], gsem[b]
        ).wait()

    def start_store(j, b):
        pltpu.async_copy(
            tile_v.at[b, :, :, pl.ds(0, _C)], o_hbm.at[j, :, wid], ssem[b]
        )

    def wait_store(b):
        pltpu.make_async_copy(
            tile_v.at[b, :, :, pl.ds(0, _C)], o_hbm.at[0, :, wid], ssem[b]
        ).wait()

    def transpose_chunk(b):
        rb, tb = rows_v.at[b], tile_v.at[b]

        def il_loop(t, ilv):
            for u in range(4):
                il = t * 4 + u
                for q in range(4):
                    v = rb[il, pl.ds(16 * q, 16)]
                    plsc.store_scatter(tb, [qv[q], dl_vec, ilv + u], v)
            return ilv + 4

        lax.fori_loop(0, _C // 4, il_loop, iot * 0)

    def substep(j, b, do_prefetch, do_store_wait):
        if do_prefetch:
            start_gather(j + _K, (b + _K) % _NBUF)
        wait_gather(b)
        if do_store_wait:
            wait_store(b)
        transpose_chunk(b)
        start_store(j, b)

    for b in range(_K):
        start_gather(b, b)

    for b in range(_NBUF):
        substep(b, b, True, False)

    def group(g, carry):
        for b in range(_NBUF):
            substep(g * _NBUF + b, b, True, True)
        return carry

    lax.fori_loop(1, _NGROUP - 1, group, 0)

    for b in range(_NBUF):
        substep((_NGROUP - 1) * _NBUF + b, b, b < _NBUF - _K, True)

    for b in range(_NBUF):
        wait_store(b)


_mesh = plsc.VectorSubcoreMesh(
    core_axis_name="c", subcore_axis_name="s",
    num_cores=_NC, num_subcores=_NS,
)

_gather = pl.kernel(
    _gather_body,
    out_type=jax.ShapeDtypeStruct((_NJ, 8, _NW, 8, _C), jnp.float32),
    mesh=_mesh,
    scratch_types=[
        pltpu.VMEM((_NJ // 8, 8, _C), jnp.int32),
        pltpu.VMEM((_NBUF, _C, _D), jnp.float32),
        pltpu.VMEM((_NBUF, 8, 8, 129), jnp.float32),
        *([pltpu.SemaphoreType.DMA] * (2 * _NBUF)),
    ],
    compiler_params=pltpu.CompilerParams(
        use_tc_tiling_on_sc=False, needs_layout_passes=False,
    ),
)


def kernel(x, W0, W1, W2, W3):
    wsum = _sum_tables(W0.T, W1.T, W2.T, W3.T).reshape(2 * _H, _D)
    xt4 = x.T.reshape(_NJ // 8, 8, _NW, _C).transpose(0, 2, 1, 3)
    out5 = _gather(wsum, xt4)
    return out5.transpose(2, 4, 0, 1, 3).reshape(_NI, _NJ, _D)
